# baseline (speedup 1.0000x reference)
import jax
import jax.numpy as jnp
from jax.experimental import pallas as pl
from jax.experimental.pallas import tpu as pltpu

_ROWS = 32768
_HID = 768
_EXP = 8
_B = 2048


def _gate_block(w_ref, thr_ref, x_ref, rwt_ref, st_ref, kt_ref, wn_ref):
    @pl.when(pl.program_id(0) == 0)
    def _():
        w = w_ref[...]
        wn = w / jnp.maximum(
            jnp.sqrt(jnp.sum(w * w, axis=0, keepdims=True)), 1e-12
        )
        wn_ref[...] = wn.astype(jnp.bfloat16)

    x = x_ref[...]
    ss = jnp.sum(x * x, axis=1, keepdims=True)
    xn = x / jnp.maximum(jnp.sqrt(ss), 1e-12)
    scores = jax.lax.dot_general(
        xn.astype(jnp.bfloat16), wn_ref[...],
        (((1,), (0,)), ((), ())),
        preferred_element_type=jnp.float32,
    )
    st = scores.T
    st_ref[...] = st
    maskf = (st > thr_ref[0, 0]).astype(jnp.float32)
    e = jnp.exp(st - 1.0) * maskf
    ssum = jnp.sum(e, axis=0, keepdims=True)
    cnt = jnp.sum(maskf, axis=0, keepdims=True)
    kt_ref[...] = cnt.astype(jnp.int32)
    rwt_ref[...] = jnp.where(cnt > 0.5, e / ssum, jnp.float32(0.125))


def kernel(hidden_states, sim_matrix, threshold):
    thr2 = threshold.reshape(1, 1)
    rwt, st, kt = pl.pallas_call(
        _gate_block,
        grid=(_ROWS // _B,),
        in_specs=[
            pl.BlockSpec((_HID, _EXP), lambda i: (0, 0)),
            pl.BlockSpec((1, 1), lambda i: (0, 0)),
            pl.BlockSpec((_B, _HID), lambda i: (i, 0)),
        ],
        out_specs=[
            pl.BlockSpec((_EXP, _B), lambda i: (0, i)),
            pl.BlockSpec((_EXP, _B), lambda i: (0, i)),
            pl.BlockSpec((1, _B), lambda i: (0, i)),
        ],
        out_shape=[
            jax.ShapeDtypeStruct((_EXP, _ROWS), jnp.float32),
            jax.ShapeDtypeStruct((_EXP, _ROWS), jnp.float32),
            jax.ShapeDtypeStruct((1, _ROWS), jnp.int32),
        ],
        scratch_shapes=[pltpu.VMEM((_HID, _EXP), jnp.bfloat16)],
        compiler_params=pltpu.CompilerParams(
            dimension_semantics=("arbitrary",),
        ),
    )(sim_matrix, thr2, hidden_states)
    return rwt.T, st.T, kt.reshape(_ROWS)


# speedup vs baseline: 1.0803x; 1.0803x over previous
import jax
import jax.numpy as jnp
from jax.experimental import pallas as pl
from jax.experimental.pallas import tpu as pltpu

_ROWS = 32768
_HID = 768
_EXP = 8
_B = 4096


def _gate_block(w_ref, thr_ref, x_ref, rwt_ref, st_ref, kt_ref, wn_ref):
    @pl.when(pl.program_id(0) == 0)
    def _():
        w = w_ref[...]
        wn = w / jnp.maximum(
            jnp.sqrt(jnp.sum(w * w, axis=0, keepdims=True)), 1e-12
        )
        wn_ref[...] = wn.astype(jnp.bfloat16)

    x = x_ref[...]
    ss = jnp.sum(x * x, axis=1, keepdims=True)
    xn = x / jnp.maximum(jnp.sqrt(ss), 1e-12)
    scores = jax.lax.dot_general(
        xn.astype(jnp.bfloat16), wn_ref[...],
        (((1,), (0,)), ((), ())),
        preferred_element_type=jnp.float32,
    )
    st = scores.T
    st_ref[...] = st
    maskf = (st > thr_ref[0, 0]).astype(jnp.float32)
    e = jnp.exp(st - 1.0) * maskf
    ssum = jnp.sum(e, axis=0, keepdims=True)
    cnt = jnp.sum(maskf, axis=0, keepdims=True)
    kt_ref[...] = cnt.astype(jnp.int32)
    rwt_ref[...] = jnp.where(cnt > 0.5, e / ssum, jnp.float32(0.125))


def kernel(hidden_states, sim_matrix, threshold):
    thr2 = threshold.reshape(1, 1)
    rwt, st, kt = pl.pallas_call(
        _gate_block,
        grid=(_ROWS // _B,),
        in_specs=[
            pl.BlockSpec((_HID, _EXP), lambda i: (0, 0)),
            pl.BlockSpec((1, 1), lambda i: (0, 0)),
            pl.BlockSpec((_B, _HID), lambda i: (i, 0)),
        ],
        out_specs=[
            pl.BlockSpec((_EXP, _B), lambda i: (0, i)),
            pl.BlockSpec((_EXP, _B), lambda i: (0, i)),
            pl.BlockSpec((1, _B), lambda i: (0, i)),
        ],
        out_shape=[
            jax.ShapeDtypeStruct((_EXP, _ROWS), jnp.float32),
            jax.ShapeDtypeStruct((_EXP, _ROWS), jnp.float32),
            jax.ShapeDtypeStruct((1, _ROWS), jnp.int32),
        ],
        scratch_shapes=[pltpu.VMEM((_HID, _EXP), jnp.bfloat16)],
        compiler_params=pltpu.CompilerParams(
            dimension_semantics=("arbitrary",),
        ),
    )(sim_matrix, thr2, hidden_states)
    return rwt.T, st.T, kt.reshape(_ROWS)
